# baseline (speedup 1.0000x reference)
import jax
import jax.numpy as jnp
from jax import lax
from jax.experimental import pallas as pl
from jax.experimental.pallas import tpu as pltpu
from jax.experimental.pallas import tpu_sc as plsc

_N = 1_000_000
_NC = 2
_NS = 16
_NW = _NC * _NS
_CHUNK = 31_264
_ZSUB = 4096
_NZ = 7
_ZTAIL = _CHUNK - _NZ * _ZSUB


def _sc_zero_pass(out_hbm, zv, semz):
    wid = lax.axis_index("s") * _NC + lax.axis_index("c")
    zbase = jnp.where(wid == _NW - 1, _N - _CHUNK, wid * _CHUNK)

    @plsc.parallel_loop(0, _ZSUB, 16, unroll=8)
    def _zero(i):
        zv[pl.ds(i, 16)] = jnp.zeros((16,), jnp.float32)

    cps = []
    for k in range(_NZ):
        cps.append(pltpu.async_copy(
            zv, out_hbm.at[pl.ds(zbase + k * _ZSUB, _ZSUB)], semz))
    cps.append(pltpu.async_copy(
        zv.at[pl.ds(0, _ZTAIL)],
        out_hbm.at[pl.ds(zbase + _NZ * _ZSUB, _ZTAIL)], semz))
    for cp in cps:
        cp.wait()


_sc_zero = pl.kernel(
    _sc_zero_pass,
    out_type=jax.ShapeDtypeStruct((_N,), jnp.float32),
    mesh=plsc.VectorSubcoreMesh(core_axis_name="c", subcore_axis_name="s",
                                num_cores=_NC, num_subcores=_NS),
    scratch_types=[
        pltpu.VMEM((_ZSUB,), jnp.float32),
        pltpu.SemaphoreType.DMA,
    ],
)


def _tree(parts, op):
    while len(parts) > 1:
        parts = [op(parts[i], parts[i + 1]) if i + 1 < len(parts)
                 else parts[i] for i in range(0, len(parts), 2)]
    return parts[0]


_RCH = 8_192
_NFULL = _N // _RCH
_RTAIL = _N - _NFULL * _RCH


_RBUF = 65_536
_NSTEP = 16
_LASTSZ = 16_896
_XT = _N - 15 * _RBUF - _LASTSZ


def _reduce_body(x_hbm, xt_ref, cmd_ref, b0, b1, s0, s1):
    bufs, sems = (b0, b1), (s0, s1)

    def fire(k):
        size = _RBUF if k < _NSTEP - 1 else _LASTSZ
        cp = pltpu.make_async_copy(
            x_hbm.at[pl.ds(k * _RBUF, size)],
            bufs[k % 2].at[pl.ds(0, size)], sems[k % 2])
        cp.start()
        return cp

    sums, maxs = [], []
    tail = None
    cps = {0: fire(0)}
    for k in range(_NSTEP):
        if k + 1 < _NSTEP:
            cps[k + 1] = fire(k + 1)
        cps[k].wait()
        buf = bufs[k % 2]
        nfull = 8 if k < _NSTEP - 1 else 2
        for j in range(nfull):
            v = buf[pl.ds(j * _RCH, _RCH)]
            sums.append(jnp.sum(jnp.abs(v)))
            maxs.append(jnp.max(v))
        if k == _NSTEP - 1:
            tail = buf[pl.ds(2 * _RCH, _LASTSZ - 2 * _RCH)]
    xt = xt_ref[...]
    s_tot = (_tree(sums, jnp.add) + jnp.sum(jnp.abs(tail))
             + jnp.sum(jnp.abs(xt)))
    gmx = jnp.maximum(jnp.maximum(_tree(maxs, jnp.maximum), jnp.max(tail)),
                      jnp.max(xt))

    ci = _tree([jnp.where(m == gmx, c, _NFULL)
                for c, m in enumerate(maxs)], jnp.minimum)
    coff = pl.multiple_of(jnp.minimum(ci, _NFULL - 1) * _RCH, 128)
    cp = pltpu.make_async_copy(
        x_hbm.at[pl.ds(coff, _RCH)], b0.at[pl.ds(0, _RCH)], s0)
    cp.start()
    cp.wait()
    v = b0[pl.ds(0, _RCH)]
    iot = lax.broadcasted_iota(jnp.int32, (_RCH,), 0) + coff
    gi = jnp.min(jnp.where(v == gmx, iot, _N))
    ti1 = jnp.min(jnp.where(
        tail == gmx,
        lax.broadcasted_iota(jnp.int32, (_LASTSZ - 2 * _RCH,), 0)
        + _NFULL * _RCH, _N))
    ti2 = jnp.min(jnp.where(
        xt == gmx,
        lax.broadcasted_iota(jnp.int32, (_XT,), 0) + (_N - _XT), _N))
    gi = jnp.minimum(gi, jnp.minimum(ti1, ti2))
    hp = jnp.logical_and(gmx > 0.0, gmx / s_tot > 0.0)
    wbase = jnp.minimum((gi // 128) * 128, _N - 128)
    ln = gi - wbase
    li = lax.broadcasted_iota(jnp.int32, (2, 128), 1)
    ri = lax.broadcasted_iota(jnp.int32, (2, 128), 0)
    row = jnp.where(li == ln, jnp.where(hp, 1.0, 0.0), 0.0)
    cmd_ref[...] = jnp.where(ri == 0, row, wbase.astype(jnp.float32))


_tc_reduce = pl.pallas_call(
    _reduce_body,
    out_shape=jax.ShapeDtypeStruct((2, 128), jnp.float32),
    in_specs=[pl.BlockSpec(memory_space=pl.ANY),
              pl.BlockSpec(memory_space=pltpu.VMEM)],
    out_specs=pl.BlockSpec(memory_space=pltpu.VMEM),
    scratch_shapes=[pltpu.VMEM((_RBUF,), jnp.float32),
                    pltpu.VMEM((_RBUF,), jnp.float32),
                    pltpu.SemaphoreType.DMA,
                    pltpu.SemaphoreType.DMA],
)


def _patch_body(cmd_ref, big_ref, out_ref, row_ref, sem):
    del big_ref
    wbase = pl.multiple_of(jnp.max(cmd_ref[1:2, :]).astype(jnp.int32), 128)
    row_ref[...] = cmd_ref[0:1, :]
    cp = pltpu.make_async_copy(row_ref.at[0],
                               out_ref.at[pl.ds(wbase, 128)], sem)
    cp.start()
    cp.wait()


_patch_kernel = pl.pallas_call(
    _patch_body,
    out_shape=jax.ShapeDtypeStruct((_N,), jnp.float32),
    in_specs=[pl.BlockSpec(memory_space=pltpu.VMEM),
              pl.BlockSpec(memory_space=pl.ANY)],
    out_specs=pl.BlockSpec(memory_space=pl.ANY),
    input_output_aliases={1: 0},
    scratch_shapes=[pltpu.VMEM((1, 128), jnp.float32),
                    pltpu.SemaphoreType.DMA],
)


@jax.jit
def _impl(x):
    cmd = _tc_reduce(x, x[_N - _XT:])
    zeros_oh = _sc_zero()
    return _patch_kernel(cmd, zeros_oh)


def kernel(x, neutralize):
    return _impl(x)


# speedup vs baseline: 1.2448x; 1.2448x over previous
import jax
import jax.numpy as jnp
from jax import lax
from jax.experimental import pallas as pl
from jax.experimental.pallas import tpu as pltpu
from jax.experimental.pallas import tpu_sc as plsc

_N = 1_000_000
_NC = 2
_NS = 16
_NW = _NC * _NS
_CHUNK = 31_264
_LAST_BASE = (_NW - 1) * _CHUNK
_LAST = _N - _LAST_BASE
_ZSUB = 4096
_NZ = 7
_ZTAIL = _CHUNK - _NZ * _ZSUB
_HALF = 15_632


def _sc_pass(x_hbm, out_hbm, rec_hbm, xv, zv, rv, semz, semr0, semr1):
    wid = lax.axis_index("s") * _NC + lax.axis_index("c")
    is_last = wid == _NW - 1
    base = jnp.where(is_last, _LAST_BASE, wid * _CHUNK)
    size = jnp.where(is_last, _LAST, _CHUNK)
    zbase = jnp.where(is_last, _N - _CHUNK, wid * _CHUNK)

    @plsc.parallel_loop(0, _ZSUB, 16, unroll=8)
    def _zero(i):
        zv[pl.ds(i, 16)] = jnp.zeros((16,), jnp.float32)

    zcps = []
    for k in range(_NZ):
        zcps.append(pltpu.async_copy(
            zv, out_hbm.at[pl.ds(zbase + k * _ZSUB, _ZSUB)], semz))
    zcps.append(pltpu.async_copy(
        zv.at[pl.ds(0, _ZTAIL)],
        out_hbm.at[pl.ds(zbase + _NZ * _ZSUB, _ZTAIL)], semz))

    cp0 = pltpu.async_copy(
        x_hbm.at[pl.ds(base, _HALF)], xv.at[pl.ds(0, _HALF)], semr0)
    cp1 = pltpu.async_copy(
        x_hbm.at[pl.ds(base + _HALF, size - _HALF)],
        xv.at[pl.ds(_HALF, size - _HALF)], semr1)

    lanes = lax.convert_element_type(lax.iota(jnp.int32, 16), jnp.float32)
    basef = lax.convert_element_type(base, jnp.float32)
    zeros = jnp.zeros((16,), jnp.float32)
    ninf = jnp.full((16,), -jnp.inf, jnp.float32)

    def reduce_span(lo, hi, carry):
        def body(i, c):
            (a0, a1, a2, a3, m0, m1, m2, m3, i0, i1, i2, i3, ix) = c
            v0 = xv[pl.ds(i, 16)]
            v1 = xv[pl.ds(i + 16, 16)]
            v2 = xv[pl.ds(i + 32, 16)]
            v3 = xv[pl.ds(i + 48, 16)]
            a0 = a0 + jnp.abs(v0)
            a1 = a1 + jnp.abs(v1)
            a2 = a2 + jnp.abs(v2)
            a3 = a3 + jnp.abs(v3)
            g0 = v0 > m0
            g1 = v1 > m1
            g2 = v2 > m2
            g3 = v3 > m3
            m0 = jnp.where(g0, v0, m0)
            m1 = jnp.where(g1, v1, m1)
            m2 = jnp.where(g2, v2, m2)
            m3 = jnp.where(g3, v3, m3)
            i0 = jnp.where(g0, ix, i0)
            i1 = jnp.where(g1, ix + 16.0, i1)
            i2 = jnp.where(g2, ix + 32.0, i2)
            i3 = jnp.where(g3, ix + 48.0, i3)
            return (a0, a1, a2, a3, m0, m1, m2, m3, i0, i1, i2, i3, ix + 64.0)

        return plsc.parallel_loop(lo, hi, 64, unroll=2, carry=carry)(body)

    carry0 = (zeros, zeros, zeros, zeros, ninf, ninf, ninf, ninf,
              zeros, zeros, zeros, zeros, basef + lanes)

    cp0.wait()
    c = reduce_span(0, _HALF - 16, carry0)
    cp1.wait()
    c = c[:12] + (c[12] + 16.0,)
    c = reduce_span(_HALF, size - 16, c)
    (a0, a1, a2, a3, m0, m1, m2, m3, i0, i1, i2, i3, ix) = c

    def fold_tail(off, a, m, idx, idxvec):
        v = xv[pl.ds(off, 16)]
        g = v > m
        return (a + jnp.abs(v), jnp.where(g, v, m), jnp.where(g, idxvec, idx))

    t0 = basef + lax.convert_element_type(_HALF - 16, jnp.float32) + lanes
    t1 = basef + lax.convert_element_type(size - 16, jnp.float32) + lanes
    a0, m0, i0 = fold_tail(_HALF - 16, a0, m0, i0, t0)
    a1, m1, i1 = fold_tail(size - 16, a1, m1, i1, t1)

    def merge(m_a, i_a, m_b, i_b):
        take_b = jnp.logical_or(m_b > m_a,
                                jnp.logical_and(m_b == m_a, i_b < i_a))
        return (jnp.where(take_b, m_b, m_a), jnp.where(take_b, i_b, i_a))

    acc = (a0 + a1) + (a2 + a3)
    mm0, mi0 = merge(m0, i0, m1, i1)
    mm1, mi1 = merge(m2, i2, m3, i3)
    mm, mi = merge(mm0, mi0, mm1, mi1)

    rv[pl.ds(0, 16)] = acc
    rv[pl.ds(16, 16)] = mm
    rv[pl.ds(32, 16)] = mi
    pltpu.sync_copy(rv.at[pl.ds(0, 16)], rec_hbm.at[pl.ds(wid * 16, 16)])
    pltpu.sync_copy(rv.at[pl.ds(16, 16)],
                    rec_hbm.at[pl.ds(512 + wid * 16, 16)])
    pltpu.sync_copy(rv.at[pl.ds(32, 16)],
                    rec_hbm.at[pl.ds(1024 + wid * 16, 16)])

    for cpz in zcps:
        cpz.wait()


_sc_kernel = pl.kernel(
    _sc_pass,
    out_type=(jax.ShapeDtypeStruct((_N,), jnp.float32),
              jax.ShapeDtypeStruct((3 * _NW * 16,), jnp.float32)),
    mesh=plsc.VectorSubcoreMesh(core_axis_name="c", subcore_axis_name="s",
                                num_cores=_NC, num_subcores=_NS),
    scratch_types=[
        pltpu.VMEM((_CHUNK,), jnp.float32),
        pltpu.VMEM((_ZSUB,), jnp.float32),
        pltpu.VMEM((48,), jnp.float32),
        pltpu.SemaphoreType.DMA,
        pltpu.SemaphoreType.DMA,
        pltpu.SemaphoreType.DMA,
    ],
)


def _patch_body(rec_ref, big_ref, out_ref, row_ref, sem):
    del big_ref
    r = rec_ref[...]
    s_tot = jnp.sum(r[0:512])
    mx = r[512:1024]
    mi = r[1024:1536]
    gmx = jnp.max(mx)
    gif = jnp.min(jnp.where(mx == gmx, mi, 2.0e9))
    hp = jnp.logical_and(gmx > 0.0, gmx / s_tot > 0.0)
    gi = gif.astype(jnp.int32)
    wbase = pl.multiple_of(jnp.minimum((gi // 128) * 128, _N - 128), 128)
    ln = gi - wbase
    li = lax.broadcasted_iota(jnp.int32, (1, 128), 1)
    row_ref[...] = jnp.where(li == ln, jnp.where(hp, 1.0, 0.0), 0.0)
    cp = pltpu.make_async_copy(row_ref.at[0],
                               out_ref.at[pl.ds(wbase, 128)], sem)
    cp.start()
    cp.wait()


_patch_kernel = pl.pallas_call(
    _patch_body,
    out_shape=jax.ShapeDtypeStruct((_N,), jnp.float32),
    in_specs=[pl.BlockSpec(memory_space=pltpu.VMEM),
              pl.BlockSpec(memory_space=pl.ANY)],
    out_specs=pl.BlockSpec(memory_space=pl.ANY),
    input_output_aliases={1: 0},
    scratch_shapes=[pltpu.VMEM((1, 128), jnp.float32),
                    pltpu.SemaphoreType.DMA],
)


@jax.jit
def _impl(x):
    zeros_oh, recs = _sc_kernel(x)
    return _patch_kernel(recs, zeros_oh)


def kernel(x, neutralize):
    return _impl(x)
